# baseline (speedup 1.0000x reference)
import functools

import jax
import jax.numpy as jnp
from jax import lax
from jax.experimental import pallas as pl
from jax.experimental.pallas import tpu as pltpu
from jax.experimental.pallas import tpu_sc as plsc

NUM_BITS = 10
EMBED_DIM = 32
ROWS = 16384
COLS = 100
VOCAB = 1024

NC = 2
NS = 16
L = 16
NW = NC * NS
RW = ROWS // NW
RB = RW // 128
CC = 20
NCH = COLS // CC


def _sc_body(flagsT, tableT, embT, table_v, flags_v, emb_v,
             in_sem, out_sem0, out_sem1):
    wid = lax.axis_index("s") * NC + lax.axis_index("c")
    r0 = wid * RW
    out_sems = (out_sem0, out_sem1)

    pltpu.sync_copy(tableT, table_v)
    pltpu.async_copy(
        flagsT.at[pl.ds(0, CC), pl.ds(r0, RW)], flags_v.at[0], in_sem
    ).wait()

    for k in range(NCH):
        in_slot = k % 2
        if k + 1 < NCH:
            pltpu.async_copy(
                flagsT.at[pl.ds((k + 1) * CC, CC), pl.ds(r0, RW)],
                flags_v.at[(k + 1) % 2], in_sem)

        @pl.loop(0, CC, step=2)
        def _col_pair(c2, _k=k, _slot=in_slot):
            for b in range(2):
                cl = c2 + b
                c = _k * CC + cl
                sem = out_sems[b]

                def drain():
                    pltpu.make_async_copy(
                        emb_v.at[b], embT.at[c, :, pl.ds(wid * RB, RB)],
                        sem).wait()

                if _k == 0:
                    @pl.when(c2 + b >= 2)
                    def _():
                        drain()
                else:
                    drain()

                @plsc.parallel_loop(0, RW // L, 1, unroll=3)
                def _vec(vi):
                    f = flags_v[_slot, cl, pl.ds(vi * L, L)]
                    for d in range(EMBED_DIM):
                        emb_v[b, d // 8, vi // 8, d % 8,
                              pl.ds((vi % 8) * L, L)] = plsc.load_gather(
                            table_v, [f + d * VOCAB])

                pltpu.async_copy(
                    emb_v.at[b], embT.at[c, :, pl.ds(wid * RB, RB)], sem)

        if k + 1 < NCH:
            pltpu.make_async_copy(
                flagsT.at[pl.ds((k + 1) * CC, CC), pl.ds(r0, RW)],
                flags_v.at[(k + 1) % 2], in_sem
            ).wait()

    for b in range(2):
        c = COLS - 2 + b
        pltpu.make_async_copy(
            emb_v.at[b], embT.at[c, :, pl.ds(wid * RB, RB)],
            out_sems[b]).wait()


def _tc_body(flagsT_ref, validT_ref, bitsT_ref):
    f = flagsT_ref[...]
    validT_ref[...] = (f == 0).astype(jnp.float32)
    shifts = lax.broadcasted_iota(jnp.int32, (NUM_BITS, 1, 1), 0)
    bitsT_ref[...] = ((f[None] >> shifts) & 1).astype(jnp.float32)


TC_GRID = 32
TC_BLK = ROWS // TC_GRID


@jax.jit
def _featurize(flagsT, tableT):
    mesh = plsc.VectorSubcoreMesh(core_axis_name="c", subcore_axis_name="s")
    emb5 = functools.partial(
        pl.kernel,
        out_type=jax.ShapeDtypeStruct(
            (COLS, EMBED_DIM // 8, ROWS // 128, 8, 128), jnp.float32),
        mesh=mesh,
        compiler_params=pltpu.CompilerParams(
            needs_layout_passes=False, use_tc_tiling_on_sc=False),
        scratch_types=[
            pltpu.VMEM((VOCAB * EMBED_DIM,), jnp.float32),
            pltpu.VMEM((2, CC, RW), jnp.int32),
            pltpu.VMEM((2, EMBED_DIM // 8, RB, 8, 128), jnp.float32),
            pltpu.SemaphoreType.DMA,
            pltpu.SemaphoreType.DMA,
            pltpu.SemaphoreType.DMA,
        ],
    )(_sc_body)(flagsT.reshape(COLS, ROWS), tableT)

    validT, bitsT = pl.pallas_call(
        _tc_body,
        grid=(TC_GRID,),
        in_specs=[pl.BlockSpec((COLS, TC_BLK), lambda i: (0, i))],
        out_specs=[
            pl.BlockSpec((COLS, TC_BLK), lambda i: (0, i)),
            pl.BlockSpec((NUM_BITS, COLS, TC_BLK), lambda i: (0, 0, i)),
        ],
        out_shape=[
            jax.ShapeDtypeStruct((COLS, ROWS), jnp.float32),
            jax.ShapeDtypeStruct((NUM_BITS, COLS, ROWS), jnp.float32),
        ],
    )(flagsT)
    return validT, emb5, bitsT


def kernel(qc_flags, table):
    flagsT = qc_flags.astype(jnp.int32).T
    tableT = table.T.reshape(-1)
    validT, emb5, bitsT = _featurize(flagsT, tableT)
    emb = emb5.transpose(2, 4, 0, 1, 3).reshape(ROWS, COLS, EMBED_DIM)
    return (
        validT.T,
        emb,
        bitsT.transpose(2, 1, 0),
    )


# speedup vs baseline: 1.4753x; 1.4753x over previous
import functools

import jax
import jax.numpy as jnp
from jax import lax
from jax.experimental import pallas as pl
from jax.experimental.pallas import tpu as pltpu
from jax.experimental.pallas import tpu_sc as plsc

NUM_BITS = 10
EMBED_DIM = 32
ROWS = 16384
COLS = 100
VOCAB = 1024

NC = 2
NS = 16
L = 16
NW = NC * NS
RW = ROWS // NW
RB = RW // 128
CC = 20
NCH = COLS // CC


def _sc_body(flagsT, tableT, embT, table_v, flags_v, emb_v,
             in_sem, out_sem0, out_sem1):
    wid = lax.axis_index("s") * NC + lax.axis_index("c")
    r0 = wid * RW
    out_sems = (out_sem0, out_sem1)

    pltpu.sync_copy(tableT, table_v)
    pltpu.async_copy(
        flagsT.at[pl.ds(0, CC), pl.ds(r0, RW)], flags_v.at[0], in_sem
    ).wait()

    for k in range(NCH):
        in_slot = k % 2
        if k + 1 < NCH:
            pltpu.async_copy(
                flagsT.at[pl.ds((k + 1) * CC, CC), pl.ds(r0, RW)],
                flags_v.at[(k + 1) % 2], in_sem)

        @pl.loop(0, CC, step=2)
        def _col_pair(c2, _k=k, _slot=in_slot):
            for b in range(2):
                cl = c2 + b
                c = _k * CC + cl
                sem = out_sems[b]

                def drain():
                    pltpu.make_async_copy(
                        emb_v.at[b], embT.at[c, :, pl.ds(wid * RB, RB)],
                        sem).wait()

                if _k == 0:
                    @pl.when(c2 + b >= 2)
                    def _():
                        drain()
                else:
                    drain()

                @plsc.parallel_loop(0, RW // L, 1, unroll=2)
                def _vec(vi):
                    f = flags_v[_slot, cl, pl.ds(vi * L, L)]
                    for d in range(EMBED_DIM):
                        emb_v[b, d // 8, vi // 8, d % 8,
                              pl.ds((vi % 8) * L, L)] = plsc.load_gather(
                            table_v, [f + d * VOCAB])

                pltpu.async_copy(
                    emb_v.at[b], embT.at[c, :, pl.ds(wid * RB, RB)], sem)

        if k + 1 < NCH:
            pltpu.make_async_copy(
                flagsT.at[pl.ds((k + 1) * CC, CC), pl.ds(r0, RW)],
                flags_v.at[(k + 1) % 2], in_sem
            ).wait()

    for b in range(2):
        c = COLS - 2 + b
        pltpu.make_async_copy(
            emb_v.at[b], embT.at[c, :, pl.ds(wid * RB, RB)],
            out_sems[b]).wait()


def _tc_body(flagsT_ref, validT_ref, bitsT_ref):
    f = flagsT_ref[...]
    validT_ref[...] = (f == 0).astype(jnp.float32)
    shifts = lax.broadcasted_iota(jnp.int32, (NUM_BITS, 1, 1), 0)
    bitsT_ref[...] = ((f[None] >> shifts) & 1).astype(jnp.float32)


TC_GRID = 32
TC_BLK = ROWS // TC_GRID


@jax.jit
def _featurize(flagsT, tableT):
    mesh = plsc.VectorSubcoreMesh(core_axis_name="c", subcore_axis_name="s")
    emb5 = functools.partial(
        pl.kernel,
        out_type=jax.ShapeDtypeStruct(
            (COLS, EMBED_DIM // 8, ROWS // 128, 8, 128), jnp.float32),
        mesh=mesh,
        compiler_params=pltpu.CompilerParams(
            needs_layout_passes=False, use_tc_tiling_on_sc=False),
        scratch_types=[
            pltpu.VMEM((VOCAB * EMBED_DIM,), jnp.float32),
            pltpu.VMEM((2, CC, RW), jnp.int32),
            pltpu.VMEM((2, EMBED_DIM // 8, RB, 8, 128), jnp.float32),
            pltpu.SemaphoreType.DMA,
            pltpu.SemaphoreType.DMA,
            pltpu.SemaphoreType.DMA,
        ],
    )(_sc_body)(flagsT.reshape(COLS, ROWS), tableT)

    validT, bitsT = pl.pallas_call(
        _tc_body,
        grid=(TC_GRID,),
        in_specs=[pl.BlockSpec((COLS, TC_BLK), lambda i: (0, i))],
        out_specs=[
            pl.BlockSpec((COLS, TC_BLK), lambda i: (0, i)),
            pl.BlockSpec((NUM_BITS, COLS, TC_BLK), lambda i: (0, 0, i)),
        ],
        out_shape=[
            jax.ShapeDtypeStruct((COLS, ROWS), jnp.float32),
            jax.ShapeDtypeStruct((NUM_BITS, COLS, ROWS), jnp.float32),
        ],
    )(flagsT)
    return validT, emb5, bitsT


def kernel(qc_flags, table):
    flagsT = qc_flags.astype(jnp.int32).T
    tableT = table.T.reshape(-1)
    validT, emb5, bitsT = _featurize(flagsT, tableT)
    emb = emb5.transpose(2, 4, 0, 1, 3).reshape(ROWS, COLS, EMBED_DIM)
    return (
        validT.T,
        emb,
        bitsT.transpose(2, 1, 0),
    )
